# baseline (device time: 106076 ns/iter reference)
import jax
import jax.numpy as jnp
from jax import lax
from jax.experimental import pallas as pl
from jax.experimental.pallas import tpu as pltpu

N_DEV = 16
B = 2
SQ = 256
D_MODEL = 768
H_LOC = 8
DH = 64
D_LOC = H_LOC * DH
ROWS = B * SQ
CH = ROWS // N_DEV
HOPS = N_DEV - 1


def kernel(x, Wq, Wo, Wk, Wv):
    def body(x_ref, wq_ref, wo_ref, wk_ref, wv_ref, out_ref,
             a_ref, part_ref, send_rs, recv_rs, red_ref, recv_ag,
             send_sem, rs_sems, ag_sems):
        my = lax.axis_index("i")
        left = (my + N_DEV - 1) % N_DEV
        right = (my + 1) % N_DEV

        barrier = pltpu.get_barrier_semaphore()
        for nbr in (left, right):
            pl.semaphore_signal(barrier, inc=1, device_id=(nbr,),
                                device_id_type=pl.DeviceIdType.MESH)
        pl.semaphore_wait(barrier, 2)

        x2d = x_ref[...].reshape(ROWS, D_MODEL)
        q = jnp.dot(x2d, wq_ref[...], preferred_element_type=jnp.float32)
        k = jnp.dot(x2d, wk_ref[...], preferred_element_type=jnp.float32)
        v = jnp.dot(x2d, wv_ref[...], preferred_element_type=jnp.float32)

        for b in range(B):
            for h in range(H_LOC):
                r0 = b * SQ
                c0 = h * DH
                qh = q[r0:r0 + SQ, c0:c0 + DH]
                kh = k[r0:r0 + SQ, c0:c0 + DH]
                vh = v[r0:r0 + SQ, c0:c0 + DH]
                s = lax.dot_general(
                    qh, kh, (((1,), (1,)), ((), ())),
                    preferred_element_type=jnp.float32) * 0.125
                m = jnp.max(s, axis=1, keepdims=True)
                p = jnp.exp(s - m)
                l = jnp.sum(p, axis=1, keepdims=True)
                o = jnp.dot(p, vh, preferred_element_type=jnp.float32) / l
                a_ref[r0:r0 + SQ, c0:c0 + DH] = o

        part_ref[...] = jnp.dot(a_ref[...], wo_ref[...],
                                preferred_element_type=jnp.float32)

        for h in range(HOPS):
            idx = (my + N_DEV - h) % N_DEV
            chunk = part_ref[pl.ds(idx * CH, CH), :]
            if h == 0:
                send_rs[h] = chunk
            else:
                send_rs[h] = chunk + recv_rs[h - 1]
            rdma = pltpu.make_async_remote_copy(
                src_ref=send_rs.at[h],
                dst_ref=recv_rs.at[h],
                send_sem=send_sem,
                recv_sem=rs_sems.at[h],
                device_id=(right,),
                device_id_type=pl.DeviceIdType.MESH,
            )
            rdma.start()
            rdma.wait()

        red_idx = (my + 1) % N_DEV
        red_ref[...] = part_ref[pl.ds(red_idx * CH, CH), :] + recv_rs[HOPS - 1]
        out_ref[pl.ds(red_idx * CH, CH), :] = red_ref[...]

        for h in range(HOPS):
            src = red_ref if h == 0 else recv_ag.at[h - 1]
            rdma = pltpu.make_async_remote_copy(
                src_ref=src,
                dst_ref=recv_ag.at[h],
                send_sem=send_sem,
                recv_sem=ag_sems.at[h],
                device_id=(right,),
                device_id_type=pl.DeviceIdType.MESH,
            )
            rdma.start()
            rdma.wait()
            origin = (my + N_DEV - h) % N_DEV
            out_ref[pl.ds(origin * CH, CH), :] = recv_ag[h]

    out = pl.pallas_call(
        body,
        out_shape=jax.ShapeDtypeStruct((ROWS, D_MODEL), jnp.float32),
        in_specs=[pl.BlockSpec(memory_space=pltpu.VMEM)] * 5,
        out_specs=pl.BlockSpec(memory_space=pltpu.VMEM),
        scratch_shapes=[
            pltpu.VMEM((ROWS, D_LOC), jnp.float32),
            pltpu.VMEM((ROWS, D_MODEL), jnp.float32),
            pltpu.VMEM((HOPS, CH, D_MODEL), jnp.float32),
            pltpu.VMEM((HOPS, CH, D_MODEL), jnp.float32),
            pltpu.VMEM((CH, D_MODEL), jnp.float32),
            pltpu.VMEM((HOPS, CH, D_MODEL), jnp.float32),
            pltpu.SemaphoreType.DMA,
            pltpu.SemaphoreType.DMA((HOPS,)),
            pltpu.SemaphoreType.DMA((HOPS,)),
        ],
        compiler_params=pltpu.CompilerParams(collective_id=0),
    )(x, Wq, Wo, Wk, Wv)
    return out.reshape(B, SQ, D_MODEL)


# device time: 51788 ns/iter; 2.0483x vs baseline; 2.0483x over previous
import jax
import jax.numpy as jnp
from jax import lax
from jax.experimental import pallas as pl
from jax.experimental.pallas import tpu as pltpu

N_DEV = 16
B = 2
SQ = 256
D_MODEL = 768
H_LOC = 8
DH = 64
D_LOC = H_LOC * DH
ROWS = B * SQ
CH = ROWS // N_DEV


def _tree_sum(vals):
    while len(vals) > 1:
        nxt = [a + b for a, b in zip(vals[::2], vals[1::2])]
        if len(vals) % 2:
            nxt.append(vals[-1])
        vals = nxt
    return vals[0]


def kernel(x, Wq, Wo, Wk, Wv):
    def body(x_ref, wq_ref, wo_ref, wk_ref, wv_ref, out_ref,
             a_ref, part_ref, red_ref, rs_buf, ag_buf,
             rs_send_sems, rs_recv_sems, ag_send_sems, ag_recv_sems):
        my = lax.axis_index("i")

        barrier = pltpu.get_barrier_semaphore()
        for o in range(1, N_DEV):
            pl.semaphore_signal(barrier, inc=1,
                                device_id=((my + o) % N_DEV,),
                                device_id_type=pl.DeviceIdType.MESH)

        x2d = x_ref[...].reshape(ROWS, D_MODEL)
        q = jnp.dot(x2d, wq_ref[...], preferred_element_type=jnp.float32)
        k = jnp.dot(x2d, wk_ref[...], preferred_element_type=jnp.float32)
        v = jnp.dot(x2d, wv_ref[...], preferred_element_type=jnp.float32)

        for b in range(B):
            for h in range(H_LOC):
                r0 = b * SQ
                c0 = h * DH
                qh = q[r0:r0 + SQ, c0:c0 + DH]
                kh = k[r0:r0 + SQ, c0:c0 + DH]
                vh = v[r0:r0 + SQ, c0:c0 + DH]
                s = lax.dot_general(
                    qh, kh, (((1,), (1,)), ((), ())),
                    preferred_element_type=jnp.float32) * 0.125
                m = jnp.max(s, axis=1, keepdims=True)
                p = jnp.exp(s - m)
                l = jnp.sum(p, axis=1, keepdims=True)
                o = jnp.dot(p, vh, preferred_element_type=jnp.float32) / l
                a_ref[r0:r0 + SQ, c0:c0 + DH] = o

        part = jnp.dot(a_ref[...], wo_ref[...],
                       preferred_element_type=jnp.float32)
        part_ref[...] = part.reshape(N_DEV, CH, D_MODEL)

        pl.semaphore_wait(barrier, N_DEV - 1)

        rs_rdmas = []
        for o in range(1, N_DEV):
            d = (my + o) % N_DEV
            rdma = pltpu.make_async_remote_copy(
                src_ref=part_ref.at[d],
                dst_ref=rs_buf.at[my],
                send_sem=rs_send_sems.at[d],
                recv_sem=rs_recv_sems.at[my],
                device_id=(d,),
                device_id_type=pl.DeviceIdType.MESH,
            )
            rdma.start()
            rs_rdmas.append(rdma)
        loc = pltpu.make_async_copy(
            part_ref.at[my], rs_buf.at[my], rs_recv_sems.at[my])
        loc.start()

        for s in range(N_DEV):
            pltpu.make_async_remote_copy(
                src_ref=rs_buf.at[s], dst_ref=rs_buf.at[s],
                send_sem=rs_send_sems.at[s], recv_sem=rs_recv_sems.at[s],
                device_id=(my,), device_id_type=pl.DeviceIdType.MESH,
            ).wait_recv()

        red_ref[...] = _tree_sum([rs_buf[s] for s in range(N_DEV)])
        for r in rs_rdmas:
            r.wait_send()

        ag_rdmas = []
        for o in range(1, N_DEV):
            d = (my + o) % N_DEV
            rdma = pltpu.make_async_remote_copy(
                src_ref=red_ref,
                dst_ref=ag_buf.at[my],
                send_sem=ag_send_sems.at[d],
                recv_sem=ag_recv_sems.at[my],
                device_id=(d,),
                device_id_type=pl.DeviceIdType.MESH,
            )
            rdma.start()
            ag_rdmas.append(rdma)
        loc2 = pltpu.make_async_copy(red_ref, ag_buf.at[my],
                                     ag_recv_sems.at[my])
        loc2.start()

        for s in range(N_DEV):
            pltpu.make_async_remote_copy(
                src_ref=ag_buf.at[s], dst_ref=ag_buf.at[s],
                send_sem=ag_send_sems.at[s], recv_sem=ag_recv_sems.at[s],
                device_id=(my,), device_id_type=pl.DeviceIdType.MESH,
            ).wait_recv()
            out_ref[s] = ag_buf[s]

        for r in ag_rdmas:
            r.wait_send()

    out = pl.pallas_call(
        body,
        out_shape=jax.ShapeDtypeStruct((N_DEV, CH, D_MODEL), jnp.float32),
        in_specs=[pl.BlockSpec(memory_space=pltpu.VMEM)] * 5,
        out_specs=pl.BlockSpec(memory_space=pltpu.VMEM),
        scratch_shapes=[
            pltpu.VMEM((ROWS, D_LOC), jnp.float32),
            pltpu.VMEM((N_DEV, CH, D_MODEL), jnp.float32),
            pltpu.VMEM((CH, D_MODEL), jnp.float32),
            pltpu.VMEM((N_DEV, CH, D_MODEL), jnp.float32),
            pltpu.VMEM((N_DEV, CH, D_MODEL), jnp.float32),
            pltpu.SemaphoreType.DMA((N_DEV,)),
            pltpu.SemaphoreType.DMA((N_DEV,)),
            pltpu.SemaphoreType.DMA((N_DEV,)),
            pltpu.SemaphoreType.DMA((N_DEV,)),
        ],
        compiler_params=pltpu.CompilerParams(collective_id=0),
    )(x, Wq, Wo, Wk, Wv)
    return out.reshape(B, SQ, D_MODEL)
